# Initial kernel scaffold; baseline (speedup 1.0000x reference)
import jax
import jax.numpy as jnp
from jax.experimental import pallas as pl


def kernel(x, edge_index, c1_tc1_w, c1_tc1_b, c1_cheb_w, c1_cheb_b, c1_tc2_w, c1_tc2_b, c1_bn_w, c1_bn_b, c2_tc1_w, c2_tc1_b, c2_cheb_w, c2_cheb_b, c2_tc2_w, c2_tc2_b, c2_bn_w, c2_bn_b):
    raise NotImplementedError("write your pallas kernel here")


# speedup vs baseline: 112.7146x; 112.7146x over previous
import functools

import jax
import jax.numpy as jnp
import numpy as np
from jax import lax
from jax.experimental import pallas as pl
from jax.experimental.pallas import tpu as pltpu
from jax.experimental.pallas import tpu_sc as plsc

N = 10000
E = 320000
EPS = 1e-5

NC = 2
NS = 16
NW = NC * NS
EPW = E // NW
CHUNK = 80
NCHUNK = EPW // CHUNK
NP = 10240
RPT = NP // NS
DEGW = 16

NB = 2000
GRID = N // NB

_mesh = functools.partial(plsc.VectorSubcoreMesh,
                          core_axis_name="c", subcore_axis_name="s",
                          num_cores=NC, num_subcores=NS)
_sc_params = functools.partial(pltpu.CompilerParams, use_tc_tiling_on_sc=False)


def _sc_deg(src_r, zeros):
    def body(src_ref, zeros_ref, out_ref, src_v, ones_v, acc_sh):
        c = lax.axis_index("c")
        s = lax.axis_index("s")
        w = c * NS + s
        pltpu.sync_copy(src_ref.at[w], src_v)
        for k in range(CHUNK):
            ones_v[k, :] = jnp.ones((DEGW,), jnp.float32)
        pltpu.sync_copy(zeros_ref.at[pl.ds(s * RPT, RPT)],
                        acc_sh.at[pl.ds(s * RPT, RPT)])
        plsc.subcore_barrier()

        def chunk(j, carry):
            pltpu.sync_copy(ones_v, acc_sh.at[src_v.at[j]], add=True)
            return carry
        lax.fori_loop(0, NCHUNK, chunk, 0)
        plsc.subcore_barrier()
        pltpu.sync_copy(acc_sh.at[pl.ds(s * RPT, RPT)],
                        out_ref.at[c, pl.ds(s * RPT, RPT)])

    return pl.kernel(
        body,
        out_type=jax.ShapeDtypeStruct((NC, NP, DEGW), jnp.float32),
        mesh=_mesh(),
        compiler_params=_sc_params(),
        scratch_types=[
            pltpu.VMEM((NCHUNK, CHUNK), jnp.int32),
            pltpu.VMEM((CHUNK, DEGW), jnp.float32),
            pltpu.VMEM_SHARED((NP, DEGW), jnp.float32),
        ],
    )(src_r, zeros)


def _sc_prop(tables, src_r, dst_r, zeros, C):
    NH = len(tables)

    def body(*refs):
        t_refs = refs[:NH]
        src_ref, dst_ref, zeros_ref, out_ref = refs[NH:NH + 4]
        src_v, dst_v, buf_v, acc_sh, sem = refs[NH + 4:]
        c = lax.axis_index("c")
        s = lax.axis_index("s")
        w = c * NS + s
        pltpu.sync_copy(src_ref.at[w], src_v)
        pltpu.sync_copy(dst_ref.at[w], dst_v)
        for h, t_ref in enumerate(t_refs):
            pltpu.sync_copy(zeros_ref.at[pl.ds(s * RPT, RPT)],
                            acc_sh.at[pl.ds(s * RPT, RPT)])
            plsc.subcore_barrier()

            def chunk(j, carry):
                pltpu.async_copy(t_ref.at[src_v.at[j]], buf_v, sem).wait()
                pltpu.sync_copy(buf_v, acc_sh.at[dst_v.at[j]], add=True)
                return carry
            lax.fori_loop(0, NCHUNK, chunk, 0)
            plsc.subcore_barrier()
            pltpu.sync_copy(acc_sh.at[pl.ds(s * RPT, RPT)],
                            out_ref.at[h, c, pl.ds(s * RPT, RPT)])
            if h + 1 < NH:
                plsc.subcore_barrier()

    return pl.kernel(
        body,
        out_type=jax.ShapeDtypeStruct((NH, NC, NP, C), jnp.float32),
        mesh=_mesh(),
        compiler_params=_sc_params(),
        scratch_types=[
            pltpu.VMEM((NCHUNK, CHUNK), jnp.int32),
            pltpu.VMEM((NCHUNK, CHUNK), jnp.int32),
            pltpu.VMEM((CHUNK, C), jnp.float32),
            pltpu.VMEM_SHARED((NP, C), jnp.float32),
            pltpu.SemaphoreType.DMA,
        ],
    )(*tables, src_r, dst_r, zeros)


def _band(W, B, t_out):
    _, out_c, in_c, _, ks = W.shape
    t_in = t_out + ks - 1
    eyes = np.stack([np.eye(t_in, t_out, -k, dtype=np.float32)
                     for k in range(ks)])
    mats, biases = [], []
    for g in range(3):
        Wk = W[g][:, :, 0, :]
        M = jnp.einsum('kab,oik->aibo', eyes, Wk)
        mats.append(M.reshape(t_in * in_c, t_out * out_c))
        biases.append(jnp.tile(B[g], t_out))
    return jnp.concatenate(mats, axis=1), jnp.concatenate(biases)[None, :]


def _cheb_cat(chw, chb, t):
    eye = np.eye(t, dtype=np.float32)
    blocks = [jnp.kron(eye, chw[k]) for k in range(3)]
    return jnp.concatenate(blocks, axis=0), jnp.tile(chb, t)[None, :]


def _dot(a, b):
    return jnp.dot(a, b, preferred_element_type=jnp.float32,
                   precision=lax.Precision.HIGHEST)


def _gate(h, f):
    p, q, r = h[:, :f], h[:, f:2 * f], h[:, 2 * f:3 * f]
    return jax.nn.relu(p * jax.nn.sigmoid(q) + r)


def _full(shape):
    zero = (0,) * len(shape)
    return pl.BlockSpec(shape, lambda i, _z=zero: _z)


def _rows(f):
    return pl.BlockSpec((NB, f), lambda i: (i, 0))


def _pp_spec(NH, C):
    return pl.BlockSpec((NH, NC, NB, C), lambda i: (0, 0, i, 0))


def _tc1(xT, degT, w1, b1):
    def body(x_ref, deg_ref, w_ref, b_ref, z_ref, za_ref, zb_ref, dis_ref):
        deg = jnp.sum(deg_ref[...], axis=1, keepdims=True)
        dis = jnp.where(deg > 0, lax.rsqrt(jnp.maximum(deg, 1e-12)), 0.0)
        h = _dot(x_ref[...], w_ref[...]) + b_ref[...]
        z = _gate(h, 160)
        zs = z * dis
        z_ref[...] = z
        za_ref[...] = zs[:, :80]
        zb_ref[...] = zs[:, 80:]
        dis_ref[...] = dis

    return pl.pallas_call(
        body,
        grid=(GRID,),
        in_specs=[_rows(12), _rows(2), _full(w1.shape), _full(b1.shape)],
        out_specs=[_rows(160), _rows(80), _rows(80), _rows(1)],
        out_shape=[jax.ShapeDtypeStruct((N, 160), jnp.float32),
                   jax.ShapeDtypeStruct((N, 80), jnp.float32),
                   jax.ShapeDtypeStruct((N, 80), jnp.float32),
                   jax.ShapeDtypeStruct((N, 1), jnp.float32)],
    )(xT, degT, w1, b1)


def _mid(pp, dis, C, NH):
    def body(pp_ref, dis_ref, p1_ref, *t_refs):
        dis = dis_ref[...]
        s = jnp.concatenate([pp_ref[h, 0] + pp_ref[h, 1]
                             for h in range(NH)], axis=1) if NH > 1 else (
            pp_ref[0, 0] + pp_ref[0, 1])
        p1 = -dis * s
        p1_ref[...] = p1
        for h in range(NH):
            t_refs[h][...] = dis * p1[:, h * C:(h + 1) * C]

    return pl.pallas_call(
        body,
        grid=(GRID,),
        in_specs=[_pp_spec(NH, C), _rows(1)],
        out_specs=[_rows(NH * C)] + [_rows(C)] * NH,
        out_shape=([jax.ShapeDtypeStruct((N, NH * C), jnp.float32)]
                   + [jax.ShapeDtypeStruct((N, C), jnp.float32)] * NH),
    )(pp, dis)


def _stage_mid(z, p1, pp, dis, bn, wc, cb, w2, b2, w3, b3):
    def body(z_ref, p1_ref, pp_ref, dis_ref, bn_ref, wc_ref, cb_ref,
             w2_ref, b2_ref, w3_ref, b3_ref, z2_ref, zs2_ref):
        z = z_ref[...]
        dis = dis_ref[...]
        s2 = jnp.concatenate([pp_ref[0, 0] + pp_ref[0, 1],
                              pp_ref[1, 0] + pp_ref[1, 1]], axis=1)
        p2 = -dis * s2
        u = jnp.concatenate([z, p1_ref[...], 2.0 * p2 - z], axis=1)
        g = jax.nn.relu(_dot(u, wc_ref[...]) + cb_ref[...])
        h = _dot(g, w2_ref[...]) + b2_ref[...]
        t2 = _gate(h, 256)
        m = jnp.mean(t2, axis=1, keepdims=True)
        v = jnp.mean((t2 - m) ** 2, axis=1, keepdims=True)
        bnp = bn_ref[...]
        tn = (t2 - m) * lax.rsqrt(v + EPS) * bnp[:, 0:1] + bnp[:, 1:2]
        e = jnp.where(tn > 0, tn, jnp.exp(jnp.minimum(tn, 0.0)) - 1.0)
        h3 = _dot(e, w3_ref[...]) + b3_ref[...]
        z2 = _gate(h3, 96)
        z2_ref[...] = z2
        zs2_ref[...] = z2 * dis

    return pl.pallas_call(
        body,
        grid=(GRID,),
        in_specs=[_rows(160), _rows(160), _pp_spec(2, 80),
                  _rows(1), _rows(2),
                  _full(wc.shape), _full(cb.shape), _full(w2.shape),
                  _full(b2.shape), _full(w3.shape), _full(b3.shape)],
        out_specs=[_rows(96), _rows(96)],
        out_shape=[jax.ShapeDtypeStruct((N, 96), jnp.float32),
                   jax.ShapeDtypeStruct((N, 96), jnp.float32)],
    )(z, p1, pp, dis, bn, wc, cb, w2, b2, w3, b3)


def _stage_final(z, p1, pp, dis, bn, wc, cb, w4, b4):
    def body(z_ref, p1_ref, pp_ref, dis_ref, bn_ref, wc_ref, cb_ref,
             w4_ref, b4_ref, out_ref):
        z = z_ref[...]
        dis = dis_ref[...]
        p2 = -dis * (pp_ref[0, 0] + pp_ref[0, 1])
        u = jnp.concatenate([z, p1_ref[...], 2.0 * p2 - z], axis=1)
        g = jax.nn.relu(_dot(u, wc_ref[...]) + cb_ref[...])
        h = _dot(g, w4_ref[...]) + b4_ref[...]
        t4 = _gate(h, 128)
        m = jnp.mean(t4, axis=1, keepdims=True)
        v = jnp.mean((t4 - m) ** 2, axis=1, keepdims=True)
        bnp = bn_ref[...]
        out_ref[...] = ((t4 - m) * lax.rsqrt(v + EPS) * bnp[:, 0:1]
                        + bnp[:, 1:2])

    return pl.pallas_call(
        body,
        grid=(GRID,),
        in_specs=[_rows(96), _rows(96), _pp_spec(1, 96),
                  _rows(1), _rows(2),
                  _full(wc.shape), _full(cb.shape),
                  _full(w4.shape), _full(b4.shape)],
        out_specs=_rows(128),
        out_shape=jax.ShapeDtypeStruct((N, 128), jnp.float32),
    )(z, p1, pp, dis, bn, wc, cb, w4, b4)


def kernel(x, edge_index,
           c1_tc1_w, c1_tc1_b, c1_cheb_w, c1_cheb_b, c1_tc2_w, c1_tc2_b,
           c1_bn_w, c1_bn_b,
           c2_tc1_w, c2_tc1_b, c2_cheb_w, c2_cheb_b, c2_tc2_w, c2_tc2_b,
           c2_bn_w, c2_bn_b):
    src_r = edge_index[0].reshape(NW, NCHUNK, CHUNK)
    dst_r = edge_index[1].reshape(NW, NCHUNK, CHUNK)
    xT = jnp.transpose(x[0, :, :, 0])

    w1, b1 = _band(c1_tc1_w, c1_tc1_b, 10)
    wc1, cb1 = _cheb_cat(c1_cheb_w, c1_cheb_b, 10)
    w2, b2 = _band(c1_tc2_w, c1_tc2_b, 8)
    w3, b3 = _band(c2_tc1_w, c2_tc1_b, 6)
    wc2, cb2 = _cheb_cat(c2_cheb_w, c2_cheb_b, 6)
    w4, b4 = _band(c2_tc2_w, c2_tc2_b, 4)
    bn1 = jnp.stack([c1_bn_w, c1_bn_b], axis=1)
    bn2 = jnp.stack([c2_bn_w, c2_bn_b], axis=1)

    zeros_deg = jnp.zeros((NP, DEGW), jnp.float32)
    zeros80 = jnp.zeros((NP, 80), jnp.float32)
    zeros96 = jnp.zeros((NP, 96), jnp.float32)

    degp = _sc_deg(src_r, zeros_deg)
    degT = jnp.swapaxes(degp[:, :N, 0], 0, 1)

    z1, zsa, zsb, dis = _tc1(xT, degT, w1, b1)
    pp = _sc_prop([zsa, zsb], src_r, dst_r, zeros80, 80)
    p1, ta, tb = _mid(pp, dis, 80, 2)
    pp2 = _sc_prop([ta, tb], src_r, dst_r, zeros80, 80)

    z2, zs2 = _stage_mid(z1, p1, pp2, dis, bn1, wc1, cb1, w2, b2, w3, b3)

    pp3 = _sc_prop([zs2], src_r, dst_r, zeros96, 96)
    p1b, t4 = _mid(pp3, dis, 96, 1)
    pp4 = _sc_prop([t4], src_r, dst_r, zeros96, 96)
    out = _stage_final(z2, p1b, pp4, dis, bn2, wc2, cb2, w4, b4)

    return jnp.transpose(out.reshape(N, 4, 32), (1, 0, 2))[None]


# speedup vs baseline: 198.0315x; 1.7569x over previous
import functools

import jax
import jax.numpy as jnp
import numpy as np
from jax import lax
from jax.experimental import pallas as pl
from jax.experimental.pallas import tpu as pltpu
from jax.experimental.pallas import tpu_sc as plsc

N = 10000
E = 320000
EPS = 1e-5

NC = 2
NS = 16
NW = NC * NS
EPW = E // NW
CHUNK = 80
NCHUNK = EPW // CHUNK
NP = 10240
RPT = NP // NS
DEGW = 16

NB = 2000
GRID = N // NB

_mesh = functools.partial(plsc.VectorSubcoreMesh,
                          core_axis_name="c", subcore_axis_name="s",
                          num_cores=NC, num_subcores=NS)
_sc_params = functools.partial(pltpu.CompilerParams, use_tc_tiling_on_sc=False)


def _sc_deg(src_r, zeros):
    def body(src_ref, zeros_ref, out_ref, src_v, ones_v, acc_sh):
        c = lax.axis_index("c")
        s = lax.axis_index("s")
        w = c * NS + s
        pltpu.sync_copy(src_ref.at[w], src_v)
        for k in range(CHUNK):
            ones_v[k, :] = jnp.ones((DEGW,), jnp.float32)
        pltpu.sync_copy(zeros_ref.at[pl.ds(s * RPT, RPT)],
                        acc_sh.at[pl.ds(s * RPT, RPT)])
        plsc.subcore_barrier()

        def chunk(j, carry):
            pltpu.sync_copy(ones_v, acc_sh.at[src_v.at[j]], add=True)
            return carry
        lax.fori_loop(0, NCHUNK, chunk, 0)
        plsc.subcore_barrier()
        pltpu.sync_copy(acc_sh.at[pl.ds(s * RPT, RPT)],
                        out_ref.at[c, pl.ds(s * RPT, RPT)])

    return pl.kernel(
        body,
        out_type=jax.ShapeDtypeStruct((NC, NP, DEGW), jnp.float32),
        mesh=_mesh(),
        compiler_params=_sc_params(),
        scratch_types=[
            pltpu.VMEM((NCHUNK, CHUNK), jnp.int32),
            pltpu.VMEM((CHUNK, DEGW), jnp.float32),
            pltpu.VMEM_SHARED((NP, DEGW), jnp.float32),
        ],
    )(src_r, zeros)


NBUF = 5
NOUT = NCHUNK // NBUF


def _sc_prop(tables, src_r, dst_r, zeros, C):
    NH = len(tables)

    def body(*refs):
        t_refs = refs[:NH]
        src_ref, dst_ref, zeros_ref, out_ref = refs[NH:NH + 4]
        scratch = refs[NH + 4:]
        src_v, dst_v = scratch[0], scratch[1]
        bufs = scratch[2:2 + NBUF]
        acc_sh = scratch[2 + NBUF]
        sems = scratch[3 + NBUF:3 + NBUF + NBUF]
        c = lax.axis_index("c")
        s = lax.axis_index("s")
        w = c * NS + s
        pltpu.sync_copy(src_ref.at[w], src_v)
        pltpu.sync_copy(dst_ref.at[w], dst_v)
        for h, t_ref in enumerate(t_refs):
            for b in range(NBUF):
                pltpu.async_copy(t_ref.at[src_v.at[b]], bufs[b], sems[b])
            pltpu.sync_copy(zeros_ref.at[pl.ds(s * RPT, RPT)],
                            acc_sh.at[pl.ds(s * RPT, RPT)])
            plsc.subcore_barrier()

            def outer(g, carry):
                for b in range(NBUF):
                    j = g * NBUF + b
                    pltpu.make_async_copy(t_ref.at[src_v.at[j]], bufs[b],
                                          sems[b]).wait()
                    pltpu.sync_copy(bufs[b], acc_sh.at[dst_v.at[j]],
                                    add=True)
                    pltpu.async_copy(t_ref.at[src_v.at[j + NBUF]], bufs[b],
                                     sems[b])
                return carry
            lax.fori_loop(0, NOUT - 1, outer, 0)
            for b in range(NBUF):
                j = (NOUT - 1) * NBUF + b
                pltpu.make_async_copy(t_ref.at[src_v.at[j]], bufs[b],
                                      sems[b]).wait()
                pltpu.sync_copy(bufs[b], acc_sh.at[dst_v.at[j]], add=True)
            plsc.subcore_barrier()
            pltpu.sync_copy(acc_sh.at[pl.ds(s * RPT, RPT)],
                            out_ref.at[h, c, pl.ds(s * RPT, RPT)])
            if h + 1 < NH:
                plsc.subcore_barrier()

    return pl.kernel(
        body,
        out_type=jax.ShapeDtypeStruct((NH, NC, NP, C), jnp.float32),
        mesh=_mesh(),
        compiler_params=_sc_params(),
        scratch_types=(
            [pltpu.VMEM((NCHUNK, CHUNK), jnp.int32),
             pltpu.VMEM((NCHUNK, CHUNK), jnp.int32)]
            + [pltpu.VMEM((CHUNK, C), jnp.float32)] * NBUF
            + [pltpu.VMEM_SHARED((NP, C), jnp.float32)]
            + [pltpu.SemaphoreType.DMA] * NBUF
        ),
    )(*tables, src_r, dst_r, zeros)


def _band(W, B, t_out):
    _, out_c, in_c, _, ks = W.shape
    t_in = t_out + ks - 1
    eyes = np.stack([np.eye(t_in, t_out, -k, dtype=np.float32)
                     for k in range(ks)])
    mats, biases = [], []
    for g in range(3):
        Wk = W[g][:, :, 0, :]
        M = jnp.einsum('kab,oik->aibo', eyes, Wk)
        mats.append(M.reshape(t_in * in_c, t_out * out_c))
        biases.append(jnp.tile(B[g], t_out))
    return jnp.concatenate(mats, axis=1), jnp.concatenate(biases)[None, :]


def _cheb_cat(chw, chb, t):
    eye = np.eye(t, dtype=np.float32)
    blocks = [jnp.kron(eye, chw[k]) for k in range(3)]
    return jnp.concatenate(blocks, axis=0), jnp.tile(chb, t)[None, :]


def _dot(a, b):
    return jnp.dot(a, b, preferred_element_type=jnp.float32,
                   precision=lax.Precision.HIGHEST)


def _gate(h, f):
    p, q, r = h[:, :f], h[:, f:2 * f], h[:, 2 * f:3 * f]
    return jax.nn.relu(p * jax.nn.sigmoid(q) + r)


def _full(shape):
    zero = (0,) * len(shape)
    return pl.BlockSpec(shape, lambda i, _z=zero: _z)


def _rows(f):
    return pl.BlockSpec((NB, f), lambda i: (i, 0))


def _pp_spec(NH, C):
    return pl.BlockSpec((NH, NC, NB, C), lambda i: (0, 0, i, 0))


def _tc1(xT, degT, w1, b1):
    def body(x_ref, deg_ref, w_ref, b_ref, z_ref, za_ref, zb_ref, dis_ref):
        deg = jnp.sum(deg_ref[...], axis=1, keepdims=True)
        dis = jnp.where(deg > 0, lax.rsqrt(jnp.maximum(deg, 1e-12)), 0.0)
        h = _dot(x_ref[...], w_ref[...]) + b_ref[...]
        z = _gate(h, 160)
        zs = z * dis
        z_ref[...] = z
        za_ref[...] = zs[:, :80]
        zb_ref[...] = zs[:, 80:]
        dis_ref[...] = dis

    return pl.pallas_call(
        body,
        grid=(GRID,),
        in_specs=[_rows(12), _rows(2), _full(w1.shape), _full(b1.shape)],
        out_specs=[_rows(160), _rows(80), _rows(80), _rows(1)],
        out_shape=[jax.ShapeDtypeStruct((N, 160), jnp.float32),
                   jax.ShapeDtypeStruct((N, 80), jnp.float32),
                   jax.ShapeDtypeStruct((N, 80), jnp.float32),
                   jax.ShapeDtypeStruct((N, 1), jnp.float32)],
    )(xT, degT, w1, b1)


def _mid(pp, dis, C, NH):
    def body(pp_ref, dis_ref, p1_ref, *t_refs):
        dis = dis_ref[...]
        s = jnp.concatenate([pp_ref[h, 0] + pp_ref[h, 1]
                             for h in range(NH)], axis=1) if NH > 1 else (
            pp_ref[0, 0] + pp_ref[0, 1])
        p1 = -dis * s
        p1_ref[...] = p1
        for h in range(NH):
            t_refs[h][...] = dis * p1[:, h * C:(h + 1) * C]

    return pl.pallas_call(
        body,
        grid=(GRID,),
        in_specs=[_pp_spec(NH, C), _rows(1)],
        out_specs=[_rows(NH * C)] + [_rows(C)] * NH,
        out_shape=([jax.ShapeDtypeStruct((N, NH * C), jnp.float32)]
                   + [jax.ShapeDtypeStruct((N, C), jnp.float32)] * NH),
    )(pp, dis)


def _stage_mid(z, p1, pp, dis, bn, wc, cb, w2, b2, w3, b3):
    def body(z_ref, p1_ref, pp_ref, dis_ref, bn_ref, wc_ref, cb_ref,
             w2_ref, b2_ref, w3_ref, b3_ref, z2_ref, zs2_ref):
        z = z_ref[...]
        dis = dis_ref[...]
        s2 = jnp.concatenate([pp_ref[0, 0] + pp_ref[0, 1],
                              pp_ref[1, 0] + pp_ref[1, 1]], axis=1)
        p2 = -dis * s2
        u = jnp.concatenate([z, p1_ref[...], 2.0 * p2 - z], axis=1)
        g = jax.nn.relu(_dot(u, wc_ref[...]) + cb_ref[...])
        h = _dot(g, w2_ref[...]) + b2_ref[...]
        t2 = _gate(h, 256)
        m = jnp.mean(t2, axis=1, keepdims=True)
        v = jnp.mean((t2 - m) ** 2, axis=1, keepdims=True)
        bnp = bn_ref[...]
        tn = (t2 - m) * lax.rsqrt(v + EPS) * bnp[:, 0:1] + bnp[:, 1:2]
        e = jnp.where(tn > 0, tn, jnp.exp(jnp.minimum(tn, 0.0)) - 1.0)
        h3 = _dot(e, w3_ref[...]) + b3_ref[...]
        z2 = _gate(h3, 96)
        z2_ref[...] = z2
        zs2_ref[...] = z2 * dis

    return pl.pallas_call(
        body,
        grid=(GRID,),
        in_specs=[_rows(160), _rows(160), _pp_spec(2, 80),
                  _rows(1), _rows(2),
                  _full(wc.shape), _full(cb.shape), _full(w2.shape),
                  _full(b2.shape), _full(w3.shape), _full(b3.shape)],
        out_specs=[_rows(96), _rows(96)],
        out_shape=[jax.ShapeDtypeStruct((N, 96), jnp.float32),
                   jax.ShapeDtypeStruct((N, 96), jnp.float32)],
    )(z, p1, pp, dis, bn, wc, cb, w2, b2, w3, b3)


def _stage_final(z, p1, pp, dis, bn, wc, cb, w4, b4):
    def body(z_ref, p1_ref, pp_ref, dis_ref, bn_ref, wc_ref, cb_ref,
             w4_ref, b4_ref, out_ref):
        z = z_ref[...]
        dis = dis_ref[...]
        p2 = -dis * (pp_ref[0, 0] + pp_ref[0, 1])
        u = jnp.concatenate([z, p1_ref[...], 2.0 * p2 - z], axis=1)
        g = jax.nn.relu(_dot(u, wc_ref[...]) + cb_ref[...])
        h = _dot(g, w4_ref[...]) + b4_ref[...]
        t4 = _gate(h, 128)
        m = jnp.mean(t4, axis=1, keepdims=True)
        v = jnp.mean((t4 - m) ** 2, axis=1, keepdims=True)
        bnp = bn_ref[...]
        out_ref[...] = ((t4 - m) * lax.rsqrt(v + EPS) * bnp[:, 0:1]
                        + bnp[:, 1:2])

    return pl.pallas_call(
        body,
        grid=(GRID,),
        in_specs=[_rows(96), _rows(96), _pp_spec(1, 96),
                  _rows(1), _rows(2),
                  _full(wc.shape), _full(cb.shape),
                  _full(w4.shape), _full(b4.shape)],
        out_specs=_rows(128),
        out_shape=jax.ShapeDtypeStruct((N, 128), jnp.float32),
    )(z, p1, pp, dis, bn, wc, cb, w4, b4)


def kernel(x, edge_index,
           c1_tc1_w, c1_tc1_b, c1_cheb_w, c1_cheb_b, c1_tc2_w, c1_tc2_b,
           c1_bn_w, c1_bn_b,
           c2_tc1_w, c2_tc1_b, c2_cheb_w, c2_cheb_b, c2_tc2_w, c2_tc2_b,
           c2_bn_w, c2_bn_b):
    src_r = edge_index[0].reshape(NW, NCHUNK, CHUNK)
    dst_r = edge_index[1].reshape(NW, NCHUNK, CHUNK)
    xT = jnp.transpose(x[0, :, :, 0])

    w1, b1 = _band(c1_tc1_w, c1_tc1_b, 10)
    wc1, cb1 = _cheb_cat(c1_cheb_w, c1_cheb_b, 10)
    w2, b2 = _band(c1_tc2_w, c1_tc2_b, 8)
    w3, b3 = _band(c2_tc1_w, c2_tc1_b, 6)
    wc2, cb2 = _cheb_cat(c2_cheb_w, c2_cheb_b, 6)
    w4, b4 = _band(c2_tc2_w, c2_tc2_b, 4)
    bn1 = jnp.stack([c1_bn_w, c1_bn_b], axis=1)
    bn2 = jnp.stack([c2_bn_w, c2_bn_b], axis=1)

    zeros_deg = jnp.zeros((NP, DEGW), jnp.float32)
    zeros80 = jnp.zeros((NP, 80), jnp.float32)
    zeros96 = jnp.zeros((NP, 96), jnp.float32)

    degp = _sc_deg(src_r, zeros_deg)
    degT = jnp.swapaxes(degp[:, :N, 0], 0, 1)

    z1, zsa, zsb, dis = _tc1(xT, degT, w1, b1)
    pp = _sc_prop([zsa, zsb], src_r, dst_r, zeros80, 80)
    p1, ta, tb = _mid(pp, dis, 80, 2)
    pp2 = _sc_prop([ta, tb], src_r, dst_r, zeros80, 80)

    z2, zs2 = _stage_mid(z1, p1, pp2, dis, bn1, wc1, cb1, w2, b2, w3, b3)

    pp3 = _sc_prop([zs2], src_r, dst_r, zeros96, 96)
    p1b, t4 = _mid(pp3, dis, 96, 1)
    pp4 = _sc_prop([t4], src_r, dst_r, zeros96, 96)
    out = _stage_final(z2, p1b, pp4, dis, bn2, wc2, cb2, w4, b4)

    return jnp.transpose(out.reshape(N, 4, 32), (1, 0, 2))[None]


# speedup vs baseline: 217.3400x; 1.0975x over previous
import functools

import jax
import jax.numpy as jnp
import numpy as np
from jax import lax
from jax.experimental import pallas as pl
from jax.experimental.pallas import tpu as pltpu
from jax.experimental.pallas import tpu_sc as plsc

N = 10000
E = 320000
EPS = 1e-5

NC = 2
NS = 16
NW = NC * NS
EPW = E // NW
CHUNK = 80
NCHUNK = EPW // CHUNK
NP = 10240
RPT = NP // NS
DEGW = 16

NB = 2000
GRID = N // NB

_mesh = functools.partial(plsc.VectorSubcoreMesh,
                          core_axis_name="c", subcore_axis_name="s",
                          num_cores=NC, num_subcores=NS)
_sc_params = functools.partial(pltpu.CompilerParams, use_tc_tiling_on_sc=False)


def _sc_deg(src_r, zeros):
    def body(src_ref, zeros_ref, out_ref, src_v, ones_v, acc_sh):
        c = lax.axis_index("c")
        s = lax.axis_index("s")
        w = c * NS + s
        pltpu.sync_copy(src_ref.at[w], src_v)
        for k in range(CHUNK):
            ones_v[k, :] = jnp.ones((DEGW,), jnp.float32)
        pltpu.sync_copy(zeros_ref.at[pl.ds(s * RPT, RPT)],
                        acc_sh.at[pl.ds(s * RPT, RPT)])
        plsc.subcore_barrier()

        def chunk(j, carry):
            pltpu.sync_copy(ones_v, acc_sh.at[src_v.at[j]], add=True)
            return carry
        lax.fori_loop(0, NCHUNK, chunk, 0)
        plsc.subcore_barrier()
        pltpu.sync_copy(acc_sh.at[pl.ds(s * RPT, RPT)],
                        out_ref.at[c, pl.ds(s * RPT, RPT)])

    return pl.kernel(
        body,
        out_type=jax.ShapeDtypeStruct((NC, NP, DEGW), jnp.float32),
        mesh=_mesh(),
        compiler_params=_sc_params(),
        scratch_types=[
            pltpu.VMEM((NCHUNK, CHUNK), jnp.int32),
            pltpu.VMEM((CHUNK, DEGW), jnp.float32),
            pltpu.VMEM_SHARED((NP, DEGW), jnp.float32),
        ],
    )(src_r, zeros)


def _sc_prop(tables, src_r, dst_r, zeros, C, K):
    NH = len(tables)
    RING = 2 * K
    MAIN = (NCHUNK - RING) // RING
    TAIL = (NCHUNK - RING) % RING

    def body(*refs):
        t_refs = refs[:NH]
        src_ref, dst_ref, zeros_ref, out_ref = refs[NH:NH + 4]
        scratch = refs[NH + 4:]
        src_v, dst_v = scratch[0], scratch[1]
        bufs = scratch[2:2 + RING]
        acc_sh = scratch[2 + RING]
        g_sems = scratch[3 + RING:3 + 2 * RING]
        s_sems = scratch[3 + 2 * RING:3 + 3 * RING]
        c = lax.axis_index("c")
        s = lax.axis_index("s")
        w = c * NS + s
        pltpu.sync_copy(src_ref.at[w], src_v)
        pltpu.sync_copy(dst_ref.at[w], dst_v)
        def g_start(t_ref, j, b):
            pltpu.async_copy(t_ref.at[src_v.at[j]], bufs[b], g_sems[b])

        def g_wait(t_ref, j, b):
            pltpu.make_async_copy(t_ref.at[src_v.at[j]], bufs[b],
                                  g_sems[b]).wait()

        def s_start(j, b):
            pltpu.async_copy(bufs[b], acc_sh.at[dst_v.at[j]], s_sems[b],
                             add=True)

        def s_wait(j, b):
            pltpu.make_async_copy(bufs[b], acc_sh.at[dst_v.at[j]],
                                  s_sems[b]).wait()

        for h, t_ref in enumerate(t_refs):
            for b in range(K):
                g_start(t_ref, b, b)
            pltpu.sync_copy(zeros_ref.at[pl.ds(s * RPT, RPT)],
                            acc_sh.at[pl.ds(s * RPT, RPT)])
            plsc.subcore_barrier()
            for j in range(K):
                g_wait(t_ref, j, j)
                s_start(j, j)
                g_start(t_ref, j + K, j + K)

            def visit(j, b):
                g_wait(t_ref, j, b)
                s_start(j, b)
                b2 = (b + K) % RING
                s_wait(j - K, b2)
                g_start(t_ref, j + K, b2)

            def outer(g, carry):
                for i in range(RING):
                    j = K + g * RING + i
                    visit(j, (K + i) % RING)
                return carry
            lax.fori_loop(0, MAIN, outer, 0)
            for i in range(TAIL):
                j = K + MAIN * RING + i
                visit(j, (K + i) % RING)
            for i in range(K):
                j = NCHUNK - K + i
                b = j % RING
                g_wait(t_ref, j, b)
                s_start(j, b)
            for i in range(RING):
                j = NCHUNK - RING + i
                s_wait(j, j % RING)
            plsc.subcore_barrier()
            pltpu.sync_copy(acc_sh.at[pl.ds(s * RPT, RPT)],
                            out_ref.at[h, c, pl.ds(s * RPT, RPT)])
            if h + 1 < NH:
                plsc.subcore_barrier()

    return pl.kernel(
        body,
        out_type=jax.ShapeDtypeStruct((NH, NC, NP, C), jnp.float32),
        mesh=_mesh(),
        compiler_params=_sc_params(),
        scratch_types=(
            [pltpu.VMEM((NCHUNK, CHUNK), jnp.int32),
             pltpu.VMEM((NCHUNK, CHUNK), jnp.int32)]
            + [pltpu.VMEM((CHUNK, C), jnp.float32)] * RING
            + [pltpu.VMEM_SHARED((NP, C), jnp.float32)]
            + [pltpu.SemaphoreType.DMA] * (2 * RING)
        ),
    )(*tables, src_r, dst_r, zeros)


def _band(W, B, t_out):
    _, out_c, in_c, _, ks = W.shape
    t_in = t_out + ks - 1
    eyes = np.stack([np.eye(t_in, t_out, -k, dtype=np.float32)
                     for k in range(ks)])
    mats, biases = [], []
    for g in range(3):
        Wk = W[g][:, :, 0, :]
        M = jnp.einsum('kab,oik->aibo', eyes, Wk)
        mats.append(M.reshape(t_in * in_c, t_out * out_c))
        biases.append(jnp.tile(B[g], t_out))
    return jnp.concatenate(mats, axis=1), jnp.concatenate(biases)[None, :]


def _cheb_cat(chw, chb, t):
    eye = np.eye(t, dtype=np.float32)
    blocks = [jnp.kron(eye, chw[k]) for k in range(3)]
    return jnp.concatenate(blocks, axis=0), jnp.tile(chb, t)[None, :]


def _dot(a, b):
    ah = a.astype(jnp.bfloat16)
    al = (a - ah.astype(jnp.float32)).astype(jnp.bfloat16)
    bh = b.astype(jnp.bfloat16)
    bl = (b - bh.astype(jnp.float32)).astype(jnp.bfloat16)
    d = functools.partial(jnp.dot, preferred_element_type=jnp.float32)
    return d(ah, bh) + (d(ah, bl) + d(al, bh))


def _gate(h, f):
    p, q, r = h[:, :f], h[:, f:2 * f], h[:, 2 * f:3 * f]
    return jax.nn.relu(p * jax.nn.sigmoid(q) + r)


def _full(shape):
    zero = (0,) * len(shape)
    return pl.BlockSpec(shape, lambda i, _z=zero: _z)


def _rows(f):
    return pl.BlockSpec((NB, f), lambda i: (i, 0))


def _pp_spec(NH, C):
    return pl.BlockSpec((NH, NC, NB, C), lambda i: (0, 0, i, 0))


def _tc1(xT, degT, w1, b1):
    def body(x_ref, deg_ref, w_ref, b_ref, z_ref, za_ref, zb_ref, dis_ref):
        deg = jnp.mean(deg_ref[0] + deg_ref[1], axis=1, keepdims=True)
        dis = jnp.where(deg > 0, lax.rsqrt(jnp.maximum(deg, 1e-12)), 0.0)
        h = _dot(x_ref[...], w_ref[...]) + b_ref[...]
        z = _gate(h, 160)
        zs = z * dis
        z_ref[...] = z
        za_ref[...] = zs[:, :80]
        zb_ref[...] = zs[:, 80:]
        dis_ref[...] = dis

    return pl.pallas_call(
        body,
        grid=(GRID,),
        in_specs=[_rows(12),
                  pl.BlockSpec((NC, NB, DEGW), lambda i: (0, i, 0)),
                  _full(w1.shape), _full(b1.shape)],
        out_specs=[_rows(160), _rows(80), _rows(80), _rows(1)],
        out_shape=[jax.ShapeDtypeStruct((N, 160), jnp.float32),
                   jax.ShapeDtypeStruct((N, 80), jnp.float32),
                   jax.ShapeDtypeStruct((N, 80), jnp.float32),
                   jax.ShapeDtypeStruct((N, 1), jnp.float32)],
    )(xT, degT, w1, b1)


def _mid(pp, dis, C, NH):
    def body(pp_ref, dis_ref, p1_ref, *t_refs):
        dis = dis_ref[...]
        s = jnp.concatenate([pp_ref[h, 0] + pp_ref[h, 1]
                             for h in range(NH)], axis=1) if NH > 1 else (
            pp_ref[0, 0] + pp_ref[0, 1])
        p1 = -dis * s
        p1_ref[...] = p1
        for h in range(NH):
            t_refs[h][...] = dis * p1[:, h * C:(h + 1) * C]

    return pl.pallas_call(
        body,
        grid=(GRID,),
        in_specs=[_pp_spec(NH, C), _rows(1)],
        out_specs=[_rows(NH * C)] + [_rows(C)] * NH,
        out_shape=([jax.ShapeDtypeStruct((N, NH * C), jnp.float32)]
                   + [jax.ShapeDtypeStruct((N, C), jnp.float32)] * NH),
    )(pp, dis)


def _stage_mid(z, p1, pp, dis, bn, wc, cb, w2, b2, w3, b3):
    def body(z_ref, p1_ref, pp_ref, dis_ref, bn_ref, wc_ref, cb_ref,
             w2_ref, b2_ref, w3_ref, b3_ref, z2_ref, zs2_ref):
        z = z_ref[...]
        dis = dis_ref[...]
        s2 = jnp.concatenate([pp_ref[0, 0] + pp_ref[0, 1],
                              pp_ref[1, 0] + pp_ref[1, 1]], axis=1)
        p2 = -dis * s2
        u = jnp.concatenate([z, p1_ref[...], 2.0 * p2 - z], axis=1)
        g = jax.nn.relu(_dot(u, wc_ref[...]) + cb_ref[...])
        h = _dot(g, w2_ref[...]) + b2_ref[...]
        t2 = _gate(h, 256)
        m = jnp.mean(t2, axis=1, keepdims=True)
        v = jnp.mean((t2 - m) ** 2, axis=1, keepdims=True)
        bnp = bn_ref[...]
        tn = (t2 - m) * lax.rsqrt(v + EPS) * bnp[:, 0:1] + bnp[:, 1:2]
        e = jnp.where(tn > 0, tn, jnp.exp(jnp.minimum(tn, 0.0)) - 1.0)
        h3 = _dot(e, w3_ref[...]) + b3_ref[...]
        z2 = _gate(h3, 96)
        z2_ref[...] = z2
        zs2_ref[...] = z2 * dis

    return pl.pallas_call(
        body,
        grid=(GRID,),
        in_specs=[_rows(160), _rows(160), _pp_spec(2, 80),
                  _rows(1), _rows(2),
                  _full(wc.shape), _full(cb.shape), _full(w2.shape),
                  _full(b2.shape), _full(w3.shape), _full(b3.shape)],
        out_specs=[_rows(96), _rows(96)],
        out_shape=[jax.ShapeDtypeStruct((N, 96), jnp.float32),
                   jax.ShapeDtypeStruct((N, 96), jnp.float32)],
    )(z, p1, pp, dis, bn, wc, cb, w2, b2, w3, b3)


def _stage_final(z, p1, pp, dis, bn, wc, cb, w4, b4):
    def body(z_ref, p1_ref, pp_ref, dis_ref, bn_ref, wc_ref, cb_ref,
             w4_ref, b4_ref, out_ref):
        z = z_ref[...]
        dis = dis_ref[...]
        p2 = -dis * (pp_ref[0, 0] + pp_ref[0, 1])
        u = jnp.concatenate([z, p1_ref[...], 2.0 * p2 - z], axis=1)
        g = jax.nn.relu(_dot(u, wc_ref[...]) + cb_ref[...])
        h = _dot(g, w4_ref[...]) + b4_ref[...]
        t4 = _gate(h, 128)
        m = jnp.mean(t4, axis=1, keepdims=True)
        v = jnp.mean((t4 - m) ** 2, axis=1, keepdims=True)
        bnp = bn_ref[...]
        out_ref[...] = ((t4 - m) * lax.rsqrt(v + EPS) * bnp[:, 0:1]
                        + bnp[:, 1:2])

    return pl.pallas_call(
        body,
        grid=(GRID,),
        in_specs=[_rows(96), _rows(96), _pp_spec(1, 96),
                  _rows(1), _rows(2),
                  _full(wc.shape), _full(cb.shape),
                  _full(w4.shape), _full(b4.shape)],
        out_specs=_rows(128),
        out_shape=jax.ShapeDtypeStruct((N, 128), jnp.float32),
    )(z, p1, pp, dis, bn, wc, cb, w4, b4)


def kernel(x, edge_index,
           c1_tc1_w, c1_tc1_b, c1_cheb_w, c1_cheb_b, c1_tc2_w, c1_tc2_b,
           c1_bn_w, c1_bn_b,
           c2_tc1_w, c2_tc1_b, c2_cheb_w, c2_cheb_b, c2_tc2_w, c2_tc2_b,
           c2_bn_w, c2_bn_b):
    src_r = edge_index[0].reshape(NW, NCHUNK, CHUNK)
    dst_r = edge_index[1].reshape(NW, NCHUNK, CHUNK)
    xT = jnp.transpose(x[0, :, :, 0])

    w1, b1 = _band(c1_tc1_w, c1_tc1_b, 10)
    wc1, cb1 = _cheb_cat(c1_cheb_w, c1_cheb_b, 10)
    w2, b2 = _band(c1_tc2_w, c1_tc2_b, 8)
    w3, b3 = _band(c2_tc1_w, c2_tc1_b, 6)
    wc2, cb2 = _cheb_cat(c2_cheb_w, c2_cheb_b, 6)
    w4, b4 = _band(c2_tc2_w, c2_tc2_b, 4)
    bn1 = jnp.stack([c1_bn_w, c1_bn_b], axis=1)
    bn2 = jnp.stack([c2_bn_w, c2_bn_b], axis=1)

    zeros_deg = jnp.zeros((NP, DEGW), jnp.float32)
    zeros80 = jnp.zeros((NP, 80), jnp.float32)
    zeros96 = jnp.zeros((NP, 96), jnp.float32)

    degp = _sc_deg(src_r, zeros_deg)

    z1, zsa, zsb, dis = _tc1(xT, degp, w1, b1)
    pp = _sc_prop([zsa, zsb], src_r, dst_r, zeros80, 80, 4)
    p1, ta, tb = _mid(pp, dis, 80, 2)
    pp2 = _sc_prop([ta, tb], src_r, dst_r, zeros80, 80, 4)

    z2, zs2 = _stage_mid(z1, p1, pp2, dis, bn1, wc1, cb1, w2, b2, w3, b3)

    pp3 = _sc_prop([zs2], src_r, dst_r, zeros96, 96, 3)
    p1b, t4 = _mid(pp3, dis, 96, 1)
    pp4 = _sc_prop([t4], src_r, dst_r, zeros96, 96, 3)
    out = _stage_final(z2, p1b, pp4, dis, bn2, wc2, cb2, w4, b4)

    return jnp.transpose(out.reshape(N, 4, 32), (1, 0, 2))[None]
